# baseline (speedup 1.0000x reference)
import jax
import jax.numpy as jnp
from jax.experimental import pallas as pl
from jax.experimental.pallas import tpu as pltpu


def _probe_kernel(x_ref, *refs):
    out_refs = refs[:-1]
    acc_ref = refs[-1]
    s = pl.program_id(1)

    @pl.when(s == 0)
    def _():
        acc_ref[...] = jnp.zeros_like(acc_ref)

    acc_ref[...] += jnp.sum(x_ref[...], axis=1)

    @pl.when(s == pl.num_programs(1) - 1)
    def _():
        for i, oref in enumerate(out_refs):
            oref[...] = acc_ref[:, i:i + 1]


def kernel(x, weight, mask):
    B, S, D = x.shape
    T = weight.shape[0]
    B_tile, S_tile = 32, 128
    out = pl.pallas_call(
        _probe_kernel,
        out_shape=[jax.ShapeDtypeStruct((B, 1), jnp.float32) for _ in range(T)],
        grid=(B // B_tile, S // S_tile),
        in_specs=[pl.BlockSpec((B_tile, S_tile, D), lambda b, s: (b, s, 0))],
        out_specs=[pl.BlockSpec((B_tile, 1), lambda b, s: (b, 0)) for _ in range(T)],
        scratch_shapes=[pltpu.VMEM((B_tile, D), jnp.float32)],
        compiler_params=pltpu.CompilerParams(
            dimension_semantics=("parallel", "arbitrary"),
            vmem_limit_bytes=int(2 * B_tile * S_tile * D * 4 + (8 << 20)),
        ),
    )(x)
    return {f"t{i}": out[i] for i in range(T)}


# speedup vs baseline: 1.1212x; 1.1212x over previous
import jax
import jax.numpy as jnp
from jax.experimental import pallas as pl
from jax.experimental.pallas import tpu as pltpu


def _k2(xa_ref, xb_ref, m_ref, w_ref, *out_refs):
    Sh = xa_ref.shape[1]
    v = jnp.where(m_ref[...], 0.0, 1.0)
    va, vb = v[:, :Sh], v[:, Sh:]
    acc = (jnp.sum(xa_ref[...] * va[:, :, None], axis=1)
           + jnp.sum(xb_ref[...] * vb[:, :, None], axis=1))
    cnt = jnp.sum(v, axis=1, keepdims=True)
    inv = 1.0 / jnp.maximum(cnt, 1.0)
    out = jax.lax.dot_general(
        acc * inv, w_ref[...],
        dimension_numbers=(((1,), (1,)), ((), ())),
        preferred_element_type=jnp.float32,
    )
    for i, oref in enumerate(out_refs):
        oref[...] = out[:, i:i + 1].astype(oref.dtype)


def kernel(x, weight, mask):
    B, S, D = x.shape
    T = weight.shape[0]
    B_tile = 16
    Sh = S // 2
    out = pl.pallas_call(
        _k2,
        out_shape=[jax.ShapeDtypeStruct((B, 1), jnp.float32) for _ in range(T)],
        grid=(B // B_tile,),
        in_specs=[
            pl.BlockSpec((B_tile, Sh, D), lambda b: (b, 0, 0)),
            pl.BlockSpec((B_tile, Sh, D), lambda b: (b, 1, 0)),
            pl.BlockSpec((B_tile, S), lambda b: (b, 0)),
            pl.BlockSpec((T, D), lambda b: (0, 0)),
        ],
        out_specs=[pl.BlockSpec((B_tile, 1), lambda b: (b, 0)) for _ in range(T)],
        compiler_params=pltpu.CompilerParams(
            dimension_semantics=("parallel",),
            vmem_limit_bytes=int(4 * B_tile * Sh * D * 4 + (8 << 20)),
        ),
    )(x, x, mask, weight)
    return {f"t{i}": out[i] for i in range(T)}
